# baseline (speedup 1.0000x reference)
import functools

import jax
import jax.numpy as jnp
from jax import lax
from jax.experimental import pallas as pl
from jax.experimental.pallas import tpu as pltpu
from jax.experimental.pallas import tpu_sc as plsc

NC = 2
NS = 16
NW = NC * NS

B = 4096
L = 200
D = 64
NUM_E = 1000000
DP = 128
GB = 128
CB = B // GB
NU = L * CB
PER_W = NU // NW
R = 4
FD = 2
FI = 3


def _gather_kernel(table_hbm, bt_hbm, out_hbm, idx_v, rows_v, trans_v,
                   isem, gsem, osem):
    c_ax = lax.axis_index("c")
    s_ax = lax.axis_index("s")
    wid = s_ax * NC + c_ax
    u0 = wid * PER_W
    lanes = lax.iota(jnp.int32, 16)

    def unit_lc(u):
        return u // CB, lax.rem(u, CB)

    def fire_idx(u, slot):
        l, c = unit_lc(u)
        pltpu.async_copy(bt_hbm.at[l, pl.ds(c * GB, GB)], idx_v.at[slot],
                         isem.at[slot])

    def wait_idx(slot):
        pltpu.make_async_copy(bt_hbm.at[0, pl.ds(0, GB)], idx_v.at[slot],
                              isem.at[slot]).wait()

    def fire_gather(slot):
        pltpu.async_copy(table_hbm.at[idx_v.at[slot]], rows_v.at[slot],
                         gsem.at[slot])

    def wait_gather(slot):
        pltpu.make_async_copy(table_hbm.at[idx_v.at[0]], rows_v.at[slot],
                              gsem.at[slot]).wait()

    def fire_out(u, slot):
        l, c = unit_lc(u)
        pltpu.async_copy(trans_v.at[slot],
                         out_hbm.at[l, :, pl.ds(c * GB, GB)], osem.at[slot])

    def wait_out(slot):
        pltpu.make_async_copy(trans_v.at[0],
                              out_hbm.at[0, :, pl.ds(0, GB)],
                              osem.at[slot]).wait()

    def transpose(slot):
        rows = rows_v.at[slot]
        trans = trans_v.at[slot]
        for jb in range(GB // 16):
            rr = jb * 16 + lanes
            for d in range(D):
                cc = jnp.full((16,), d, jnp.int32)
                trans[d, pl.ds(jb * 16, 16)] = plsc.load_gather(rows, [rr, cc])

    for p in range(FI):
        fire_idx(u0 + p, p)
    for p in range(FD):
        wait_idx(p)
        fire_gather(p)

    def body(t, carry):
        slot = lax.rem(t, R)

        @pl.when(t < PER_W - FI)
        def _fi():
            fire_idx(u0 + t + FI, lax.rem(t + FI, R))

        @pl.when(t < PER_W - FD)
        def _fg():
            s2 = lax.rem(t + FD, R)
            wait_idx(s2)
            fire_gather(s2)

        wait_gather(slot)

        @pl.when(t >= R)
        def _wo():
            wait_out(slot)

        transpose(slot)
        fire_out(u0 + t, slot)
        return carry

    lax.fori_loop(0, PER_W, body, 0)

    for p in range(R):
        wait_out((PER_W - R + p) % R)


@jax.jit
def _run(table_p, batch_t):
    k = functools.partial(
        pl.kernel,
        out_type=jax.ShapeDtypeStruct((L, D, B), jnp.float32),
        mesh=plsc.VectorSubcoreMesh(core_axis_name="c", subcore_axis_name="s"),
        scratch_types=[
            pltpu.VMEM((R, GB), jnp.int32),
            pltpu.VMEM((R, GB, DP), jnp.float32),
            pltpu.VMEM((R, D, GB), jnp.float32),
            pltpu.SemaphoreType.DMA((R,)),
            pltpu.SemaphoreType.DMA((R,)),
            pltpu.SemaphoreType.DMA((R,)),
        ],
        compiler_params=pltpu.CompilerParams(needs_layout_passes=False),
    )(_gather_kernel)
    return k(table_p, batch_t)


def kernel(batch, table):
    table_p = jnp.pad(table, ((0, 0), (0, DP - D)))
    batch_t = batch.T
    out = _run(table_p, batch_t)
    return jnp.transpose(out, (2, 0, 1))


# speedup vs baseline: 1.3988x; 1.3988x over previous
import functools

import jax
import jax.numpy as jnp
from jax import lax
from jax.experimental import pallas as pl
from jax.experimental.pallas import tpu as pltpu
from jax.experimental.pallas import tpu_sc as plsc

NC = 2
NS = 16
NW = NC * NS

B = 4096
L = 200
D = 64
NUM_E = 1000000
DP = 128
GB = 128
CB = B // GB
NU = L * CB
PER_W = NU // NW
R = 4
FD = 2
FI = 3


def _gather_kernel(table_hbm, bt_hbm, out_hbm, idx_v, rows_v, trans_v,
                   isem, gsem, osem):
    c_ax = lax.axis_index("c")
    s_ax = lax.axis_index("s")
    wid = s_ax * NC + c_ax
    u0 = wid * PER_W
    lanes = lax.iota(jnp.int32, 16)

    def unit_lc(u):
        return u // CB, lax.rem(u, CB)

    def fire_idx(u, slot):
        l, c = unit_lc(u)
        pltpu.async_copy(bt_hbm.at[l, pl.ds(c * GB, GB)], idx_v.at[slot],
                         isem.at[slot])

    def wait_idx(slot):
        pltpu.make_async_copy(bt_hbm.at[0, pl.ds(0, GB)], idx_v.at[slot],
                              isem.at[slot]).wait()

    def fire_gather(slot):
        pltpu.async_copy(table_hbm.at[idx_v.at[slot]], rows_v.at[slot],
                         gsem.at[slot])

    def wait_gather(slot):
        pltpu.make_async_copy(table_hbm.at[idx_v.at[0]], rows_v.at[slot],
                              gsem.at[slot]).wait()

    def fire_out(u, slot):
        l, c = unit_lc(u)
        pltpu.async_copy(trans_v.at[slot],
                         out_hbm.at[l, :, pl.ds(c * GB, GB)], osem.at[slot])

    def wait_out(slot):
        pltpu.make_async_copy(trans_v.at[0],
                              out_hbm.at[0, :, pl.ds(0, GB)],
                              osem.at[slot]).wait()

    def transpose(slot):
        rows = rows_v.at[slot]
        trans = trans_v.at[slot]
        for jb in range(GB // 16):
            rr = jb * 16 + lanes
            for d0 in range(0, D, 8):
                vals = [plsc.load_gather(
                            rows, [rr, jnp.full((16,), d0 + i, jnp.int32)])
                        for i in range(8)]
                for i in range(8):
                    trans[d0 + i, pl.ds(jb * 16, 16)] = vals[i]

    for p in range(FI):
        fire_idx(u0 + p, p)
    for p in range(FD):
        wait_idx(p)
        fire_gather(p)

    def body(t, carry):
        slot = lax.rem(t, R)

        @pl.when(t < PER_W - FI)
        def _fi():
            fire_idx(u0 + t + FI, lax.rem(t + FI, R))

        @pl.when(t < PER_W - FD)
        def _fg():
            s2 = lax.rem(t + FD, R)
            wait_idx(s2)
            fire_gather(s2)

        wait_gather(slot)

        @pl.when(t >= R)
        def _wo():
            wait_out(slot)

        transpose(slot)
        fire_out(u0 + t, slot)
        return carry

    lax.fori_loop(0, PER_W, body, 0)

    for p in range(R):
        wait_out((PER_W - R + p) % R)


@jax.jit
def _run(table_p, batch_t):
    k = functools.partial(
        pl.kernel,
        out_type=jax.ShapeDtypeStruct((L, D, B), jnp.float32),
        mesh=plsc.VectorSubcoreMesh(core_axis_name="c", subcore_axis_name="s"),
        scratch_types=[
            pltpu.VMEM((R, GB), jnp.int32),
            pltpu.VMEM((R, GB, DP), jnp.float32),
            pltpu.VMEM((R, D, GB), jnp.float32),
            pltpu.SemaphoreType.DMA((R,)),
            pltpu.SemaphoreType.DMA((R,)),
            pltpu.SemaphoreType.DMA((R,)),
        ],
        compiler_params=pltpu.CompilerParams(needs_layout_passes=False),
    )(_gather_kernel)
    return k(table_p, batch_t)


def kernel(batch, table):
    table_p = jnp.pad(table, ((0, 0), (0, DP - D)))
    batch_t = batch.T
    out = _run(table_p, batch_t)
    return jnp.transpose(out, (2, 0, 1))


# speedup vs baseline: 1.7450x; 1.2475x over previous
import functools

import jax
import jax.numpy as jnp
from jax import lax
from jax.experimental import pallas as pl
from jax.experimental.pallas import tpu as pltpu
from jax.experimental.pallas import tpu_sc as plsc

NC = 2
NS = 16
NW = NC * NS

B = 4096
L = 200
D = 64
DP = 128
TOTAL = B * L
PER_W = TOTAL // NW
G2 = 256
NCH = PER_W // G2
R = 3
FD = 2


def _gather_kernel(table_hbm, idx_hbm, out_hbm, idx_v, rows_v, gsem, osem):
    c = lax.axis_index("c")
    s = lax.axis_index("s")
    wid = s * NC + c
    pltpu.sync_copy(idx_hbm.at[wid], idx_v)
    base = wid * PER_W

    def fire_gather(j, slot):
        pltpu.async_copy(table_hbm.at[idx_v.at[pl.ds(j * G2, G2)]],
                         rows_v.at[slot], gsem.at[slot])

    def wait_gather(slot):
        pltpu.make_async_copy(table_hbm.at[idx_v.at[pl.ds(0, G2)]],
                              rows_v.at[slot], gsem.at[slot]).wait()

    def fire_out(j, slot):
        pltpu.async_copy(rows_v.at[slot],
                         out_hbm.at[pl.ds(base + j * G2, G2)], osem.at[slot])

    def wait_out(slot):
        pltpu.make_async_copy(rows_v.at[slot],
                              out_hbm.at[pl.ds(base, G2)], osem.at[slot]).wait()

    for p in range(FD):
        fire_gather(p, p)

    def body(j, carry):
        slot = lax.rem(j, R)
        wait_gather(slot)
        fire_out(j, slot)

        @pl.when(j < NCH - FD)
        def _fire_next():
            f = j + FD
            slot2 = lax.rem(f, R)

            @pl.when(f >= R)
            def _recycle():
                wait_out(slot2)

            fire_gather(f, slot2)

        return carry

    lax.fori_loop(0, NCH, body, 0)

    for p in range(R):
        wait_out((NCH - R + p) % R)


@jax.jit
def _run(table, idx2):
    k = functools.partial(
        pl.kernel,
        out_type=jax.ShapeDtypeStruct((TOTAL, DP), jnp.float32),
        mesh=plsc.VectorSubcoreMesh(core_axis_name="c", subcore_axis_name="s"),
        scratch_types=[
            pltpu.VMEM((PER_W,), jnp.int32),
            pltpu.VMEM((R, G2, DP), jnp.float32),
            pltpu.SemaphoreType.DMA((R,)),
            pltpu.SemaphoreType.DMA((R,)),
        ],
    )(_gather_kernel)
    return k(table, idx2)


def kernel(batch, table):
    idx2 = batch.reshape(NW, PER_W)
    table_p = jnp.pad(table, ((0, 0), (0, DP - D)))
    out = _run(table_p, idx2)
    return out[:, :D].reshape(B, L, D)
